# baseline (speedup 1.0000x reference)
import jax
import jax.numpy as jnp
from jax.experimental import pallas as pl
from jax.experimental.pallas import tpu as pltpu

_F32 = jnp.float32


def _softmax_rows(s):
    s = s - jnp.max(s, axis=-1, keepdims=True)
    e = jnp.exp(s)
    return e / jnp.sum(e, axis=-1, keepdims=True)


def _fused_kernel(x_ref, wqkv_ref, wog_ref, bt_ref, wt_ref, wd_ref, bd_ref,
                  wtf_ref, btf_ref, wdf_ref, bdf_ref,
                  out_t_ref, out_d_ref,
                  gxs, cat_s, h_t_s, c_t_s, h_d_s, c_d_s, hs_t, hs_d):
    T, BB, D = x_ref.shape
    TS = out_t_ref.shape[0]
    G = gxs.shape[-1]
    H = G // 4
    g = pl.program_id(1)

    @pl.when(g == 0)
    def _attention():
        h_t_s[...] = jnp.zeros_like(h_t_s)
        c_t_s[...] = jnp.zeros_like(c_t_s)
        h_d_s[...] = jnp.zeros_like(h_d_s)
        c_d_s[...] = jnp.zeros_like(c_d_s)

        w_qkv = wqkv_ref[...]
        dn = (((1,), (1,)), ((), ()))
        for j in range(BB):
            xj = x_ref[:, j, :]
            qkv = jnp.dot(xj, w_qkv, preferred_element_type=_F32)
            q_t = qkv[:, 0 * D:1 * D]
            k_t = qkv[:, 1 * D:2 * D]
            v_t = qkv[:, 2 * D:3 * D]
            q_d = qkv[:, 3 * D:4 * D]
            k_d = qkv[:, 4 * D:5 * D]
            v_d = qkv[:, 5 * D:6 * D]
            p_t = _softmax_rows(
                jax.lax.dot_general(q_d, k_t, dn, preferred_element_type=_F32))
            p_d = _softmax_rows(
                jax.lax.dot_general(q_t, k_d, dn, preferred_element_type=_F32))
            o_t = jnp.dot(p_t, v_t, preferred_element_type=_F32)
            o_d = jnp.dot(p_d, v_d, preferred_element_type=_F32)
            cat_s[:, j, :] = jnp.concatenate([o_t, o_d], axis=-1)
        cat = cat_s[...].reshape(T * BB, 2 * D)
        gxs[...] = (jnp.dot(cat, wog_ref[...], preferred_element_type=_F32)
                    + bt_ref[...])

    @pl.when(g > 0)
    def _recurrence():
        wt_a = wt_ref[:H]
        wt_b = wt_ref[H:]
        wd_a = wd_ref[:H]
        wd_b = wd_ref[H:]
        bd = bd_ref[...]

        def cell(gates, c_prev):
            i = jax.nn.sigmoid(gates[:, 0 * H:1 * H])
            f = jax.nn.sigmoid(gates[:, 1 * H:2 * H])
            gg = jnp.tanh(gates[:, 2 * H:3 * H])
            o = jax.nn.sigmoid(gates[:, 3 * H:4 * H])
            c_new = f * c_prev + i * gg
            return o * jnp.tanh(c_new), c_new

        h_t = h_t_s[...]
        c_t = c_t_s[...]
        h_d = h_d_s[...]
        c_d = c_d_s[...]
        base = (g - 1) * TS * BB
        for ts in range(TS):
            gx = gxs[pl.ds(base + ts * BB, BB), :]
            mb = jnp.dot(h_d, wt_b, preferred_element_type=_F32)
            ma = jnp.dot(h_t, wt_a, preferred_element_type=_F32)
            h_t, c_t = cell(gx + ma + mb, c_t)
            md = jnp.dot(h_d, wd_b, preferred_element_type=_F32)
            mc = jnp.dot(h_t, wd_a, preferred_element_type=_F32)
            h_d, c_d = cell(mc + md + bd, c_d)
            hs_t[pl.ds(ts * BB, BB), :] = h_t
            hs_d[pl.ds(ts * BB, BB), :] = h_d

        h_t_s[...] = h_t
        c_t_s[...] = c_t
        h_d_s[...] = h_d
        c_d_s[...] = c_d

        proj_t = (jnp.dot(hs_t[...], wtf_ref[...],
                          preferred_element_type=_F32) + btf_ref[...])
        proj_d = (jnp.dot(hs_d[...], wdf_ref[...],
                          preferred_element_type=_F32) + bdf_ref[...])
        for ts in range(TS):
            out_t_ref[ts] = proj_t[ts * BB:(ts + 1) * BB, :]
            out_d_ref[ts] = proj_d[ts * BB:(ts + 1) * BB, :]


def kernel(x, w_qkv, w_out_gates, w_gates_t, b_gates_t, w_gates_d, b_gates_d,
           net_t_w, net_t_b, net_d_w, net_d_b):
    T, B, D = x.shape
    H = net_t_w.shape[0]
    G = 4 * H
    BB = max(8, B // 2)
    TS = 8
    while T % TS:
        TS -= 1

    def full(shape):
        return pl.BlockSpec(shape, lambda i, g: (0,) * len(shape))

    out_t, out_d = pl.pallas_call(
        _fused_kernel,
        out_shape=(jax.ShapeDtypeStruct((T, B, D), _F32),
                   jax.ShapeDtypeStruct((T, B, D), _F32)),
        grid=(B // BB, 1 + T // TS),
        in_specs=[pl.BlockSpec((T, BB, D), lambda i, g: (0, i, 0)),
                  full((D, 6 * D)), full((2 * D, G)), full((1, G)),
                  full((2 * H, G)), full((2 * H, G)), full((1, G)),
                  full((H, D)), full((1, D)), full((H, D)), full((1, D))],
        out_specs=[
            pl.BlockSpec((TS, BB, D),
                         lambda i, g: (jnp.maximum(g, 1) - 1, i, 0)),
            pl.BlockSpec((TS, BB, D),
                         lambda i, g: (jnp.maximum(g, 1) - 1, i, 0))],
        scratch_shapes=[pltpu.VMEM((T * BB, G), _F32),
                        pltpu.VMEM((T, BB, 2 * D), _F32),
                        pltpu.VMEM((BB, H), _F32),
                        pltpu.VMEM((BB, H), _F32),
                        pltpu.VMEM((BB, H), _F32),
                        pltpu.VMEM((BB, H), _F32),
                        pltpu.VMEM((TS * BB, H), _F32),
                        pltpu.VMEM((TS * BB, H), _F32)],
        compiler_params=pltpu.CompilerParams(
            dimension_semantics=("parallel", "arbitrary")),
    )(x, w_qkv, w_out_gates, b_gates_t, w_gates_t, w_gates_d, b_gates_d,
      net_t_w, net_t_b, net_d_w, net_d_b)
    return out_t, out_d


# speedup vs baseline: 1.6757x; 1.6757x over previous
import jax
import jax.numpy as jnp
from jax.experimental import pallas as pl
from jax.experimental.pallas import tpu as pltpu

_BF = jnp.bfloat16
_F32 = jnp.float32


def _softmax_rows(s):
    s = s - jnp.max(s, axis=-1, keepdims=True)
    e = jnp.exp(s)
    return e / jnp.sum(e, axis=-1, keepdims=True)


def _fused_kernel(x_ref, wqkv_ref, wog_ref, bt_ref, wt_ref, wd_ref, bd_ref,
                  wtf_ref, btf_ref, wdf_ref, bdf_ref,
                  out_t_ref, out_d_ref,
                  wqkv_bf, wog_bf, cat_s, gxs,
                  h_t_s, c_t_s, h_d_s, c_d_s, hs_t, hs_d):
    T, B, D = x_ref.shape
    TS = out_t_ref.shape[0]
    G = gxs.shape[-1]
    H = G // 4
    g = pl.program_id(0)

    @pl.when(g == 0)
    def _attention():
        h_t_s[...] = jnp.zeros_like(h_t_s)
        c_t_s[...] = jnp.zeros_like(c_t_s)
        h_d_s[...] = jnp.zeros_like(h_d_s)
        c_d_s[...] = jnp.zeros_like(c_d_s)

        wqkv_bf[...] = wqkv_ref[...].astype(_BF)
        wog_bf[...] = wog_ref[...].astype(_BF)
        w_qkv = wqkv_bf[...]
        dn = (((1,), (1,)), ((), ()))
        for b in range(B):
            xb = x_ref[:, b, :].astype(_BF)
            qkv = jnp.dot(xb, w_qkv, preferred_element_type=_F32)
            q_t = qkv[:, 0 * D:1 * D]
            k_t = qkv[:, 1 * D:2 * D]
            v_t = qkv[:, 2 * D:3 * D]
            q_d = qkv[:, 3 * D:4 * D]
            k_d = qkv[:, 4 * D:5 * D]
            v_d = qkv[:, 5 * D:6 * D]
            p_t = _softmax_rows(
                jax.lax.dot_general(q_d, k_t, dn, preferred_element_type=_F32))
            p_d = _softmax_rows(
                jax.lax.dot_general(q_t, k_d, dn, preferred_element_type=_F32))
            o_t = jnp.dot(p_t, v_t, preferred_element_type=_F32)
            o_d = jnp.dot(p_d, v_d, preferred_element_type=_F32)
            cat_s[:, b, :] = jnp.concatenate([o_t, o_d], axis=-1)
        cat = cat_s[...].reshape(T * B, 2 * D).astype(_BF)
        gxs[...] = (jnp.dot(cat, wog_bf[...], preferred_element_type=_F32)
                    + bt_ref[...])

    @pl.when(g > 0)
    def _recurrence():
        wt_a = wt_ref[:H]
        wt_b = wt_ref[H:]
        wd_a = wd_ref[:H]
        wd_b = wd_ref[H:]
        bd = bd_ref[...]

        def cell(gates, c_prev):
            i = jax.nn.sigmoid(gates[:, 0 * H:1 * H])
            f = jax.nn.sigmoid(gates[:, 1 * H:2 * H])
            gg = jnp.tanh(gates[:, 2 * H:3 * H])
            o = jax.nn.sigmoid(gates[:, 3 * H:4 * H])
            c_new = f * c_prev + i * gg
            return o * jnp.tanh(c_new), c_new

        h_t = h_t_s[...]
        c_t = c_t_s[...]
        h_d = h_d_s[...]
        c_d = c_d_s[...]
        base = (g - 1) * TS * B
        for ts in range(TS):
            gx = gxs[pl.ds(base + ts * B, B), :]
            mb = jnp.dot(h_d, wt_b, preferred_element_type=_F32)
            ma = jnp.dot(h_t, wt_a, preferred_element_type=_F32)
            h_t, c_t = cell(gx + ma + mb, c_t)
            md = jnp.dot(h_d, wd_b, preferred_element_type=_F32)
            mc = jnp.dot(h_t, wd_a, preferred_element_type=_F32)
            h_d, c_d = cell(mc + md + bd, c_d)
            hs_t[pl.ds(ts * B, B), :] = h_t
            hs_d[pl.ds(ts * B, B), :] = h_d

        h_t_s[...] = h_t
        c_t_s[...] = c_t
        h_d_s[...] = h_d
        c_d_s[...] = c_d

        proj_t = (jnp.dot(hs_t[...], wtf_ref[...],
                          preferred_element_type=_F32) + btf_ref[...])
        proj_d = (jnp.dot(hs_d[...], wdf_ref[...],
                          preferred_element_type=_F32) + bdf_ref[...])
        for ts in range(TS):
            out_t_ref[ts] = proj_t[ts * B:(ts + 1) * B, :]
            out_d_ref[ts] = proj_d[ts * B:(ts + 1) * B, :]


def kernel(x, w_qkv, w_out_gates, w_gates_t, b_gates_t, w_gates_d, b_gates_d,
           net_t_w, net_t_b, net_d_w, net_d_b):
    T, B, D = x.shape
    H = net_t_w.shape[0]
    G = 4 * H
    TS = 8
    while T % TS:
        TS -= 1

    def full(shape):
        return pl.BlockSpec(shape, lambda g: (0,) * len(shape))

    out_t, out_d = pl.pallas_call(
        _fused_kernel,
        out_shape=(jax.ShapeDtypeStruct((T, B, D), _F32),
                   jax.ShapeDtypeStruct((T, B, D), _F32)),
        grid=(1 + T // TS,),
        in_specs=[full((T, B, D)),
                  full((D, 6 * D)), full((2 * D, G)), full((1, G)),
                  full((2 * H, G)), full((2 * H, G)), full((1, G)),
                  full((H, D)), full((1, D)), full((H, D)), full((1, D))],
        out_specs=[
            pl.BlockSpec((TS, B, D), lambda g: (jnp.maximum(g, 1) - 1, 0, 0)),
            pl.BlockSpec((TS, B, D), lambda g: (jnp.maximum(g, 1) - 1, 0, 0))],
        scratch_shapes=[pltpu.VMEM((D, 6 * D), _BF),
                        pltpu.VMEM((2 * D, G), _BF),
                        pltpu.VMEM((T, B, 2 * D), _F32),
                        pltpu.VMEM((T * B, G), _F32),
                        pltpu.VMEM((B, H), _F32),
                        pltpu.VMEM((B, H), _F32),
                        pltpu.VMEM((B, H), _F32),
                        pltpu.VMEM((B, H), _F32),
                        pltpu.VMEM((TS * B, H), _F32),
                        pltpu.VMEM((TS * B, H), _F32)],
        compiler_params=pltpu.CompilerParams(
            dimension_semantics=("arbitrary",)),
    )(x, w_qkv, w_out_gates, b_gates_t, w_gates_t, w_gates_d, b_gates_d,
      net_t_w, net_t_b, net_d_w, net_d_b)
    return out_t, out_d


# speedup vs baseline: 1.7774x; 1.0606x over previous
import jax
import jax.numpy as jnp
from jax.experimental import pallas as pl
from jax.experimental.pallas import tpu as pltpu

_BF = jnp.bfloat16
_F32 = jnp.float32


def _softmax_rows(s):
    s = s - jnp.max(s, axis=-1, keepdims=True)
    e = jnp.exp(s)
    return e / jnp.sum(e, axis=-1, keepdims=True)


def _fused_kernel(x_ref, wqkv_ref, wog_ref, bt_ref, wt_ref, wd_ref, bd_ref,
                  wtf_ref, btf_ref, wdf_ref, bdf_ref,
                  out_t_ref, out_d_ref,
                  wqkv_bf, wog_bf, cat_s, gxs,
                  h_t_s, c_t_s, h_d_s, c_d_s, hs_t, hs_d):
    T, B, D = x_ref.shape
    TS = out_t_ref.shape[0]
    G = gxs.shape[-1]
    H = G // 4
    g = pl.program_id(0)

    @pl.when(g == 0)
    def _attention():
        h_t_s[...] = jnp.zeros_like(h_t_s)
        c_t_s[...] = jnp.zeros_like(c_t_s)
        h_d_s[...] = jnp.zeros_like(h_d_s)
        c_d_s[...] = jnp.zeros_like(c_d_s)

        wqkv_bf[...] = wqkv_ref[...].astype(_BF)
        wog_bf[...] = wog_ref[...].astype(_BF)
        w_qkv = wqkv_bf[...]
        dn = (((1,), (1,)), ((), ()))
        vs, scores = [], []
        for b in range(B):
            xb = x_ref[:, b, :].astype(_BF)
            qkv = jnp.dot(xb, w_qkv, preferred_element_type=_F32)
            q_t = qkv[:, 0 * D:1 * D]
            k_t = qkv[:, 1 * D:2 * D]
            q_d = qkv[:, 3 * D:4 * D]
            k_d = qkv[:, 4 * D:5 * D]
            vs.append((qkv[:, 2 * D:3 * D], qkv[:, 5 * D:6 * D]))
            scores.append((
                jax.lax.dot_general(q_d, k_t, dn, preferred_element_type=_F32),
                jax.lax.dot_general(q_t, k_d, dn, preferred_element_type=_F32)))
        probs = [(_softmax_rows(s_t), _softmax_rows(s_d))
                 for s_t, s_d in scores]
        for b in range(B):
            p_t, p_d = probs[b]
            v_t, v_d = vs[b]
            o_t = jnp.dot(p_t, v_t, preferred_element_type=_F32)
            o_d = jnp.dot(p_d, v_d, preferred_element_type=_F32)
            cat_s[:, b, :] = jnp.concatenate([o_t, o_d], axis=-1)
        cat = cat_s[...].reshape(T * B, 2 * D).astype(_BF)
        gxs[...] = (jnp.dot(cat, wog_bf[...], preferred_element_type=_F32)
                    + bt_ref[...])

    @pl.when(g > 0)
    def _recurrence():
        wt_a = wt_ref[:H]
        wt_b = wt_ref[H:]
        wd_a = wd_ref[:H]
        wd_b = wd_ref[H:]
        bd = bd_ref[...]

        def cell(gates, c_prev):
            i = jax.nn.sigmoid(gates[:, 0 * H:1 * H])
            f = jax.nn.sigmoid(gates[:, 1 * H:2 * H])
            gg = jnp.tanh(gates[:, 2 * H:3 * H])
            o = jax.nn.sigmoid(gates[:, 3 * H:4 * H])
            c_new = f * c_prev + i * gg
            return o * jnp.tanh(c_new), c_new

        h_t = h_t_s[...]
        c_t = c_t_s[...]
        h_d = h_d_s[...]
        c_d = c_d_s[...]
        base = (g - 1) * TS * B
        for ts in range(TS):
            gx = gxs[pl.ds(base + ts * B, B), :]
            mb = jnp.dot(h_d, wt_b, preferred_element_type=_F32)
            ma = jnp.dot(h_t, wt_a, preferred_element_type=_F32)
            h_t, c_t = cell(gx + ma + mb, c_t)
            md = jnp.dot(h_d, wd_b, preferred_element_type=_F32)
            mc = jnp.dot(h_t, wd_a, preferred_element_type=_F32)
            h_d, c_d = cell(mc + md + bd, c_d)
            hs_t[pl.ds(ts * B, B), :] = h_t
            hs_d[pl.ds(ts * B, B), :] = h_d

        h_t_s[...] = h_t
        c_t_s[...] = c_t
        h_d_s[...] = h_d
        c_d_s[...] = c_d

        proj_t = (jnp.dot(hs_t[...], wtf_ref[...],
                          preferred_element_type=_F32) + btf_ref[...])
        proj_d = (jnp.dot(hs_d[...], wdf_ref[...],
                          preferred_element_type=_F32) + bdf_ref[...])
        for ts in range(TS):
            out_t_ref[ts] = proj_t[ts * B:(ts + 1) * B, :]
            out_d_ref[ts] = proj_d[ts * B:(ts + 1) * B, :]


def kernel(x, w_qkv, w_out_gates, w_gates_t, b_gates_t, w_gates_d, b_gates_d,
           net_t_w, net_t_b, net_d_w, net_d_b):
    T, B, D = x.shape
    H = net_t_w.shape[0]
    G = 4 * H
    TS = 16
    while T % TS:
        TS -= 1

    def full(shape):
        return pl.BlockSpec(shape, lambda g: (0,) * len(shape))

    out_t, out_d = pl.pallas_call(
        _fused_kernel,
        out_shape=(jax.ShapeDtypeStruct((T, B, D), _F32),
                   jax.ShapeDtypeStruct((T, B, D), _F32)),
        grid=(1 + T // TS,),
        in_specs=[full((T, B, D)),
                  full((D, 6 * D)), full((2 * D, G)), full((1, G)),
                  full((2 * H, G)), full((2 * H, G)), full((1, G)),
                  full((H, D)), full((1, D)), full((H, D)), full((1, D))],
        out_specs=[
            pl.BlockSpec((TS, B, D), lambda g: (jnp.maximum(g, 1) - 1, 0, 0)),
            pl.BlockSpec((TS, B, D), lambda g: (jnp.maximum(g, 1) - 1, 0, 0))],
        scratch_shapes=[pltpu.VMEM((D, 6 * D), _BF),
                        pltpu.VMEM((2 * D, G), _BF),
                        pltpu.VMEM((T, B, 2 * D), _F32),
                        pltpu.VMEM((T * B, G), _F32),
                        pltpu.VMEM((B, H), _F32),
                        pltpu.VMEM((B, H), _F32),
                        pltpu.VMEM((B, H), _F32),
                        pltpu.VMEM((B, H), _F32),
                        pltpu.VMEM((TS * B, H), _F32),
                        pltpu.VMEM((TS * B, H), _F32)],
        compiler_params=pltpu.CompilerParams(
            dimension_semantics=("arbitrary",)),
    )(x, w_qkv, w_out_gates, b_gates_t, w_gates_t, w_gates_d, b_gates_d,
      net_t_w, net_t_b, net_d_w, net_d_b)
    return out_t, out_d
